# baseline (device time: 13083 ns/iter reference)
import jax
import jax.numpy as jnp
from jax import lax
from jax.experimental import pallas as pl
from jax.experimental.pallas import tpu as pltpu

N_DEV = 4
C = 2


def kernel(x):
    m, n = x.shape
    h = m // 2
    rows = h // C
    nchunks = 2 * C

    def body(x_ref, out_ref, xv, sbuf, tbuf, r1buf, r2buf, obuf,
             cin_sems, cout_sems, s1sems, r1sems, s2sems, r2sems):
        my = lax.axis_index("i")
        p1 = my ^ 1
        p2 = 3 - my

        barrier_sem = pltpu.get_barrier_semaphore()
        for nbr in (p1, p2):
            pl.semaphore_signal(
                barrier_sem, inc=1,
                device_id=(nbr,), device_id_type=pl.DeviceIdType.MESH,
            )
        issue_order = [a * C + c for c in range(C) for a in (0, 1)]
        cin = [None] * nchunks
        for k in issue_order:
            cp = pltpu.make_async_copy(
                x_ref.at[pl.ds(k * rows, rows), :], xv.at[k], cin_sems.at[k]
            )
            cp.start()
            cin[k] = cp
        pl.semaphore_wait(barrier_sem, 2)

        r1 = [None] * nchunks
        for k in issue_order:
            cin[k].wait()
            sbuf[k] = xv[k].astype(jnp.bfloat16)
            dev = p1 if k < C else p2
            rd = pltpu.make_async_remote_copy(
                src_ref=sbuf.at[k], dst_ref=r1buf.at[k],
                send_sem=s1sems.at[k], recv_sem=r1sems.at[k],
                device_id=(dev,), device_id_type=pl.DeviceIdType.MESH,
            )
            rd.start()
            r1[k] = rd

        r2 = [None] * nchunks
        for k in issue_order:
            r1[k].wait_recv()
            tbuf[k] = sbuf[k] + r1buf[k]
            dev = p2 if k < C else p1
            rd = pltpu.make_async_remote_copy(
                src_ref=tbuf.at[k], dst_ref=r2buf.at[k],
                send_sem=s2sems.at[k], recv_sem=r2sems.at[k],
                device_id=(dev,), device_id_type=pl.DeviceIdType.MESH,
            )
            rd.start()
            r2[k] = rd

        cout = [None] * nchunks
        for k in issue_order:
            r2[k].wait_recv()
            obuf[k] = tbuf[k] + r2buf[k]
            cp = pltpu.make_async_copy(
                obuf.at[k], out_ref.at[pl.ds(k * rows, rows), :], cout_sems.at[k]
            )
            cp.start()
            cout[k] = cp

        for cp in cout:
            cp.wait()
        for rd in r1:
            rd.wait_send()
        for rd in r2:
            rd.wait_send()

    bf16_chunks = pltpu.VMEM((nchunks, rows, n), jnp.bfloat16)
    return pl.pallas_call(
        body,
        out_shape=jax.ShapeDtypeStruct((m, n), jnp.bfloat16),
        in_specs=[pl.BlockSpec(memory_space=pl.ANY)],
        out_specs=pl.BlockSpec(memory_space=pl.ANY),
        scratch_shapes=[
            pltpu.VMEM((nchunks, rows, n), jnp.float32),
            bf16_chunks,
            bf16_chunks,
            bf16_chunks,
            bf16_chunks,
            bf16_chunks,
            pltpu.SemaphoreType.DMA((nchunks,)),
            pltpu.SemaphoreType.DMA((nchunks,)),
            pltpu.SemaphoreType.DMA((nchunks,)),
            pltpu.SemaphoreType.DMA((nchunks,)),
            pltpu.SemaphoreType.DMA((nchunks,)),
            pltpu.SemaphoreType.DMA((nchunks,)),
        ],
        compiler_params=pltpu.CompilerParams(collective_id=0),
    )(x)
